# Initial kernel scaffold; baseline (speedup 1.0000x reference)
import jax
import jax.numpy as jnp
from jax.experimental import pallas as pl


def kernel(x, position_embedding):
    raise NotImplementedError("write your pallas kernel here")


# speedup vs baseline: 1.1379x; 1.1379x over previous
import functools

import jax
import jax.numpy as jnp
from jax import lax
from jax.experimental import pallas as pl
from jax.experimental.pallas import tpu as pltpu
from jax.experimental.pallas import tpu_sc as plsc

_B = 64
_C = 3
_H = 176
_K = 32
_S = 16
_NH = 10
_NW = 10
_N = _NH * _NW


def _sc_patchify(x, pe):
    mesh = plsc.VectorSubcoreMesh(core_axis_name="c", subcore_axis_name="s")

    @functools.partial(
        pl.kernel,
        mesh=mesh,
        out_type=jax.ShapeDtypeStruct((_B, _N, _C, 8, 128), jnp.float32),
        scratch_types=[
            pltpu.VMEM((_K, _H), jnp.float32),
            pltpu.VMEM((_NW, 1, 8, 128), jnp.float32),
            pltpu.VMEM((_NW, 1, 8, 128), jnp.float32),
        ],
    )
    def k(x_hbm, pe_hbm, out_hbm, xband, pev, outv):
        wid = lax.axis_index("s") * 2 + lax.axis_index("c")

        def combo_body(combo, carry):
            ci = combo // _NH
            ii = combo % _NH
            pltpu.sync_copy(
                pe_hbm.at[pl.ds(_NW * ii, _NW), pl.ds(ci, 1)], pev)

            def b_body(bb, carry2):
                b = 2 * wid + bb
                pltpu.sync_copy(
                    x_hbm.at[b, ci, pl.ds(_S * ii, _K), :], xband)

                def q_body(q, carry3):
                    for j in range(_NW):
                        for m in range(8):
                            kh = q * 4 + (m // 2)
                            half = m % 2
                            outv[j, 0, q, pl.ds(16 * m, 16)] = (
                                xband[kh, pl.ds(_S * (j + half), _S)]
                                + pev[j, 0, q, pl.ds(16 * m, 16)]
                            )
                    return carry3

                lax.fori_loop(0, 8, q_body, 0)
                pltpu.sync_copy(
                    outv,
                    out_hbm.at[b, pl.ds(_NW * ii, _NW), pl.ds(ci, 1)])
                return carry2

            lax.fori_loop(0, 2, b_body, 0)
            return carry

        lax.fori_loop(0, _C * _NH, combo_body, 0)

    return k(x, pe)


def kernel(x, position_embedding):
    pe = position_embedding.reshape(_N, _C, 8, 128)
    out = _sc_patchify(x, pe)
    return out.reshape(_B, _N, _C, _K, _K)


# speedup vs baseline: 1.5607x; 1.3715x over previous
import functools

import jax
import jax.numpy as jnp
from jax import lax
from jax.experimental import pallas as pl
from jax.experimental.pallas import tpu as pltpu
from jax.experimental.pallas import tpu_sc as plsc

_B = 64
_C = 3
_H = 176
_K = 32
_S = 16
_NH = 10
_NW = 10
_N = _NH * _NW
_NT = _C * _NH * 2


def _sc_patchify(x, pe):
    mesh = plsc.VectorSubcoreMesh(core_axis_name="c", subcore_axis_name="s")

    @functools.partial(
        pl.kernel,
        mesh=mesh,
        out_type=jax.ShapeDtypeStruct((_B, _N, _C, 8, 128), jnp.float32),
        scratch_types=[
            pltpu.VMEM((2, _K, _H), jnp.float32),
            pltpu.VMEM((2, _NW, 1, 8, 128), jnp.float32),
            pltpu.VMEM((2, _NW, 1, 8, 128), jnp.float32),
            pltpu.SemaphoreType.DMA((2,)),
            pltpu.SemaphoreType.DMA((2,)),
            pltpu.SemaphoreType.DMA((2,)),
        ],
    )
    def k(x_hbm, pe_hbm, out_hbm, xband, pev, outv, sem_in, sem_pe, sem_out):
        wid = lax.axis_index("s") * 2 + lax.axis_index("c")

        def fire_in(t):
            q = t // 2
            b = 2 * wid + (t % 2)
            ci = q // _NH
            ii = q % _NH
            pltpu.async_copy(
                x_hbm.at[b, ci, pl.ds(_S * ii, _K), :],
                xband.at[t % 2], sem_in.at[t % 2])

        def fire_pe(q):
            ci = q // _NH
            ii = q % _NH
            pltpu.async_copy(
                pe_hbm.at[pl.ds(_NW * ii, _NW), pl.ds(ci, 1)],
                pev.at[q % 2], sem_pe.at[q % 2])

        def wait_in(t):
            pltpu.make_async_copy(
                x_hbm.at[0, 0, pl.ds(0, _K), :],
                xband.at[t % 2], sem_in.at[t % 2]).wait()

        def wait_pe(q):
            pltpu.make_async_copy(
                pe_hbm.at[pl.ds(0, _NW), pl.ds(0, 1)],
                pev.at[q % 2], sem_pe.at[q % 2]).wait()

        def wait_out(t):
            pltpu.make_async_copy(
                outv.at[t % 2],
                out_hbm.at[0, pl.ds(0, _NW), pl.ds(0, 1)],
                sem_out.at[t % 2]).wait()

        fire_pe(0)
        fire_in(0)

        def body(t, carry):
            pl.when(t < _NT - 1)(lambda: fire_in(t + 1))
            pl.when(jnp.logical_and(t % 2 == 1, t < _NT - 1))(
                lambda: fire_pe((t + 1) // 2))
            wait_in(t)
            pl.when(t % 2 == 0)(lambda: wait_pe(t // 2))
            pl.when(t >= 2)(lambda: wait_out(t - 2))

            slot = t % 2
            peslot = (t // 2) % 2

            def q_body(qq, carry3):
                for j in range(_NW):
                    for m in range(8):
                        kh = qq * 4 + (m // 2)
                        half = m % 2
                        outv[slot, j, 0, qq, pl.ds(16 * m, 16)] = (
                            xband[slot, kh, pl.ds(_S * (j + half), _S)]
                            + pev[peslot, j, 0, qq, pl.ds(16 * m, 16)]
                        )
                return carry3

            lax.fori_loop(0, 8, q_body, 0)

            q = t // 2
            b = 2 * wid + (t % 2)
            ci = q // _NH
            ii = q % _NH
            pltpu.async_copy(
                outv.at[slot],
                out_hbm.at[b, pl.ds(_NW * ii, _NW), pl.ds(ci, 1)],
                sem_out.at[slot])
            return carry

        lax.fori_loop(0, _NT, body, 0)
        wait_out(_NT - 2)
        wait_out(_NT - 1)

    return k(x, pe)


def kernel(x, position_embedding):
    pe = position_embedding.reshape(_N, _C, 8, 128)
    out = _sc_patchify(x, pe)
    return out.reshape(_B, _N, _C, _K, _K)


# speedup vs baseline: 2.3416x; 1.5004x over previous
import functools

import jax
import jax.numpy as jnp
from jax import lax
from jax.experimental import pallas as pl
from jax.experimental.pallas import tpu as pltpu
from jax.experimental.pallas import tpu_sc as plsc

_B = 64
_C = 3
_H = 176
_K = 32
_S = 16
_NH = 10
_NW = 10
_N = _NH * _NW
_NT = _C * _NH * 2


def _sc_patchify(x, pe):
    mesh = plsc.VectorSubcoreMesh(core_axis_name="c", subcore_axis_name="s")

    @functools.partial(
        pl.kernel,
        mesh=mesh,
        out_type=jax.ShapeDtypeStruct((_B, _N, _C, 8, 128), jnp.float32),
        scratch_types=[
            pltpu.VMEM((2, _K, _H), jnp.float32),
            pltpu.VMEM((2, _NW, 1, 8, 128), jnp.float32),
            pltpu.VMEM((2, _NW, 1, 8, 128), jnp.float32),
            pltpu.SemaphoreType.DMA((2,)),
            pltpu.SemaphoreType.DMA((2,)),
            pltpu.SemaphoreType.DMA((2,)),
        ],
    )
    def k(x_hbm, pe_hbm, out_hbm, xband, pev, outv, sem_in, sem_pe, sem_out):
        wid = lax.axis_index("s") * 2 + lax.axis_index("c")

        def fire_in(t):
            q = t // 2
            b = 2 * wid + (t % 2)
            ci = q // _NH
            ii = q % _NH
            pltpu.async_copy(
                x_hbm.at[b, ci, pl.ds(_S * ii, _K), :],
                xband.at[t % 2], sem_in.at[t % 2])

        def fire_pe(q):
            ci = q // _NH
            ii = q % _NH
            pltpu.async_copy(
                pe_hbm.at[pl.ds(_NW * ii, _NW), pl.ds(ci, 1)],
                pev.at[q % 2], sem_pe.at[q % 2])

        def wait_in(t):
            pltpu.make_async_copy(
                x_hbm.at[0, 0, pl.ds(0, _K), :],
                xband.at[t % 2], sem_in.at[t % 2]).wait()

        def wait_pe(q):
            pltpu.make_async_copy(
                pe_hbm.at[pl.ds(0, _NW), pl.ds(0, 1)],
                pev.at[q % 2], sem_pe.at[q % 2]).wait()

        def wait_out(t):
            pltpu.make_async_copy(
                outv.at[t % 2],
                out_hbm.at[0, pl.ds(0, _NW), pl.ds(0, 1)],
                sem_out.at[t % 2]).wait()

        fire_pe(0)
        fire_in(0)

        def body(t, carry):
            pl.when(t < _NT - 1)(lambda: fire_in(t + 1))
            pl.when(jnp.logical_and(t % 2 == 1, t < _NT - 1))(
                lambda: fire_pe((t + 1) // 2))
            wait_in(t)
            pl.when(t % 2 == 0)(lambda: wait_pe(t // 2))
            pl.when(t >= 2)(lambda: wait_out(t - 2))

            slot = t % 2
            peslot = (t // 2) % 2

            @plsc.parallel_loop(0, 8, unroll=2)
            def q_body(qq):
                for j in range(_NW):
                    for m in range(8):
                        kh = qq * 4 + (m // 2)
                        half = m % 2
                        outv[slot, j, 0, qq, pl.ds(16 * m, 16)] = (
                            xband[slot, kh, pl.ds(_S * (j + half), _S)]
                            + pev[peslot, j, 0, qq, pl.ds(16 * m, 16)]
                        )

            q = t // 2
            b = 2 * wid + (t % 2)
            ci = q // _NH
            ii = q % _NH
            pltpu.async_copy(
                outv.at[slot],
                out_hbm.at[b, pl.ds(_NW * ii, _NW), pl.ds(ci, 1)],
                sem_out.at[slot])
            return carry

        lax.fori_loop(0, _NT, body, 0)
        wait_out(_NT - 2)
        wait_out(_NT - 1)

    return k(x, pe)


def kernel(x, position_embedding):
    pe = position_embedding.reshape(_N, _C, 8, 128)
    out = _sc_patchify(x, pe)
    return out.reshape(_B, _N, _C, _K, _K)
